# Initial kernel scaffold; baseline (speedup 1.0000x reference)
import jax
import jax.numpy as jnp
from jax.experimental import pallas as pl


def kernel(x, csr_idx, W1, W2):
    raise NotImplementedError("write your pallas kernel here")


# speedup vs baseline: 3.7960x; 3.7960x over previous
import functools

import jax
import jax.numpy as jnp
from jax import lax
from jax.experimental import pallas as pl
from jax.experimental.pallas import tpu as pltpu
from jax.experimental.pallas import tpu_sc as plsc

NC = 2
NS = 16
NW = NC * NS
L = 16
CHUNK = 128
BIG = 1 << 30


def _mesh():
    return plsc.VectorSubcoreMesh(
        core_axis_name="c", subcore_axis_name="s", num_cores=NC, num_subcores=NS
    )


def _iota():
    return lax.iota(jnp.int32, L)


def _bcast(s):
    return jnp.full((L,), 1, jnp.int32) * s


def _l0(x):
    return plsc.cummax(jnp.where(_iota() == 0, x, -BIG))


def _build_next_nonempty(csr_v, t_v, nseg, dynamic):
    nch = nseg // L
    cap = jnp.full((L,), nseg - 2, jnp.int32)

    if dynamic:
        @pl.loop(0, nch, init_carry=cap)
        def carry_out(i, carry):
            k = nch - 1 - i
            base = _bcast(k * L)
            a = plsc.load_gather(csr_v, [_iota() + base])
            b = plsc.load_gather(csr_v, [_iota() + base + 1])
            idx = _iota() + base
            m = jnp.where(b > a, idx, BIG)
            pmin = -plsc.cummax(-lax.rev(m, (0,)))
            tot = jnp.minimum(jnp.minimum(pmin, carry), cap)
            t_v[pl.ds(k * L, L)] = lax.rev(tot, (0,))
            return plsc.load_gather(t_v, [base])
    else:
        raise NotImplementedError("use dynamic=True: the unrolled build races "
                                  "the carry re-read with the chunk store")


def _seg_minmax_body(
    N, D, GW, x_hbm, csr_hbm, mn_hbm, mx_hbm, csr_v, t_v, xbuf, rmin, rmax, sems
):
    NV = D // L
    wid = lax.axis_index("c") * NS + lax.axis_index("s")
    g0 = wid * GW
    pltpu.sync_copy(csr_hbm.at[pl.ds(g0, GW + 48)], csr_v)
    _build_next_nonempty(csr_v, t_v, GW + 16, dynamic=True)

    v0 = csr_v[pl.ds(0, L)]
    S = v0[0]
    vE = csr_v[pl.ds(GW, L)]
    E = vE[0]
    zero16 = jnp.full((L,), 0, jnp.int32)
    target = _l0(plsc.load_gather(csr_v, [zero16]))
    bs_lo = zero16
    bs_hi = jnp.full((L,), GW + 14, jnp.int32)

    @pl.loop(0, 10, init_carry=(bs_lo, bs_hi))
    def bs(i, lh):
        lo2, hi2 = lh
        mid = (lo2 + hi2 + 1) // 2
        pv = _l0(plsc.load_gather(csr_v, [mid]))
        le = pv <= target
        return (jnp.where(le, mid, lo2), jnp.where(le, hi2, mid - 1))

    g_vec = bs[0]
    end_vec = _l0(plsc.load_gather(csr_v, [g_vec + 1]))

    nch = (E - S + CHUNK - 1) // CHUNK
    inf = jnp.full((L,), jnp.inf, jnp.float32)
    ninf = jnp.full((L,), -jnp.inf, jnp.float32)
    iota = _iota()

    def chunk_src(c):
        bc = jnp.minimum(S + c * CHUNK, N - CHUNK)
        return bc, x_hbm.at[pl.ds(bc * D, CHUNK * D)]

    @pl.when(nch > 0)
    def _():
        _, src0 = chunk_src(jnp.int32(0))
        pltpu.async_copy(src0, xbuf.at[pl.ds(0, CHUNK * D)], sems.at[0])

    @pl.loop(0, nch, init_carry=(g_vec, end_vec, (inf,) * NV, (ninf,) * NV))
    def final(c, carry):
        base = S + c * CHUNK
        cur = lax.rem(c, 2)
        nxt = 1 - cur

        @pl.when(c + 1 < nch)
        def _():
            _, srcn = chunk_src(c + 1)
            pltpu.async_copy(
                srcn, xbuf.at[pl.ds(nxt * CHUNK * D, CHUNK * D)], sems.at[nxt]
            )

        bc, src = chunk_src(c)
        pltpu.make_async_copy(
            src, xbuf.at[pl.ds(cur * CHUNK * D, CHUNK * D)], sems.at[cur]
        ).wait()
        vb = cur * CHUNK * D
        end = jnp.minimum(E, base + CHUNK)

        @pl.loop(base, end, init_carry=carry)
        def ck(r, carry2):
            g, ev, mins, maxs = carry2
            off = vb + (r - bc) * D
            evh = _l0(ev)
            gh = _l0(g)
            last = _bcast(r + 1) == evh
            tidx = jnp.clip(g + 1, 0, GW + 15)
            g2 = jnp.where(last, plsc.load_gather(t_v, [tidx]), g)
            cidx = jnp.clip(g2 + 1, 0, GW + 47)
            ev2 = plsc.load_gather(csr_v, [cidx])
            nmins = []
            nmaxs = []
            for j in range(NV):
                v = xbuf[pl.ds(off + j * L, L)]
                nmins.append(jnp.minimum(mins[j], v))
                nmaxs.append(jnp.maximum(maxs[j], v))
            gd = gh * D
            for j in range(NV):
                plsc.store_scatter(rmin, [gd + (j * L) + iota], nmins[j], mask=last)
                plsc.store_scatter(rmax, [gd + (j * L) + iota], nmaxs[j], mask=last)
            nmins = tuple(jnp.where(last, inf, m) for m in nmins)
            nmaxs = tuple(jnp.where(last, ninf, m) for m in nmaxs)
            return (g2, ev2, nmins, nmaxs)

        return ck

    del final
    pltpu.sync_copy(rmin, mn_hbm.at[pl.ds(g0 * D, GW * D)])
    pltpu.sync_copy(rmax, mx_hbm.at[pl.ds(g0 * D, GW * D)])


def _expand_body(
    N, D, GP, pb_hbm, csr_hbm, bias_hbm, csr_v, t_v, idxb, obuf, gsem, ssem
):
    R = N // NW
    wid = lax.axis_index("c") * NS + lax.axis_index("s")
    row0 = wid * R
    pltpu.sync_copy(csr_hbm.at[pl.ds(0, GP + 48)], csr_v)
    _build_next_nonempty(csr_v, t_v, GP + 16, dynamic=True)

    target = _bcast(row0)
    lo = jnp.full((L,), 0, jnp.int32)
    hi = jnp.full((L,), GP - 1, jnp.int32)

    @pl.loop(0, 15, init_carry=(lo, hi))
    def bs(i, lh):
        lo2, hi2 = lh
        mid = (lo2 + hi2 + 1) // 2
        pv = _l0(plsc.load_gather(csr_v, [mid]))
        le = pv <= target
        return (jnp.where(le, mid, lo2), jnp.where(le, hi2, mid - 1))

    g_vec = bs[0]
    end_vec = _l0(plsc.load_gather(csr_v, [g_vec + 1]))
    iota = _iota()

    n_full = R // CHUNK
    rem = R % CHUNK

    def walk16(rbase16, g, ev):
        col = jnp.full((L,), 0, jnp.int32)
        for j in range(L):
            col = jnp.where(iota == j, _l0(g), col)
            last = _bcast(rbase16 + j + 1) == _l0(ev)
            tidx = jnp.clip(g + 1, 0, GP + 15)
            g = jnp.where(last, plsc.load_gather(t_v, [tidx]), g)
            cidx = jnp.clip(g + 1, 0, GP + 47)
            ev = plsc.load_gather(csr_v, [cidx])
        return col, g, ev

    def out_slice(c):
        return bias_hbm.at[pl.ds(pl.multiple_of(row0 + c * CHUNK, 8), CHUNK)]

    @pl.loop(0, n_full, init_carry=(g_vec, end_vec))
    def gev(c, carry):
        g, ev = carry
        base = row0 + c * CHUNK
        cur = lax.rem(c, 2)

        @pl.when(c >= 2)
        def _():
            pltpu.make_async_copy(obuf.at[cur], out_slice(c - 2), ssem.at[cur]).wait()

        for k in range(CHUNK // L):
            col, g, ev = walk16(base + k * L, g, ev)
            idxb[pl.ds(cur * CHUNK + k * L, L)] = col
        pltpu.async_copy(
            pb_hbm.at[idxb.at[pl.ds(cur * CHUNK, CHUNK)]], obuf.at[cur], gsem
        ).wait()
        pltpu.async_copy(obuf.at[cur], out_slice(c), ssem.at[cur])
        return g, ev

    for tail_c in (n_full - 2, n_full - 1):
        if tail_c >= 0:
            pltpu.make_async_copy(
                obuf.at[tail_c % 2], out_slice(jnp.int32(tail_c)), ssem.at[tail_c % 2]
            ).wait()

    if rem:
        g, ev = gev
        base = row0 + n_full * CHUNK
        for k in range(rem // L):
            col, g, ev = walk16(base + k * L, g, ev)
            idxb[pl.ds(k * L, L)] = col
        pltpu.async_copy(
            pb_hbm.at[idxb.at[pl.ds(0, rem)]], obuf.at[0].at[pl.ds(0, rem)], gsem
        ).wait()
        pltpu.sync_copy(
            obuf.at[0].at[pl.ds(0, rem)],
            bias_hbm.at[pl.ds(pl.multiple_of(base, 8), rem)],
        )


def _mid_body(D, mn_ref, mx_ref, w1_ref, w2_ref, pb_ref, wst_ref, w2t_ref):
    w1 = w1_ref[...]
    w1a = w1[:, :D]
    w1b = w1[:, D : 2 * D]
    w1c = w1[:, 2 * D :]
    pb_ref[...] = jnp.dot(
        mn_ref[...], w1b.T, preferred_element_type=jnp.float32
    ) + jnp.dot(mx_ref[...], w1c.T, preferred_element_type=jnp.float32)
    wst_ref[...] = (w1a + w1b + w1c).T.astype(jnp.bfloat16)
    w2t_ref[...] = w2_ref[...].T.astype(jnp.bfloat16)


def _mlp_body(x_ref, b_ref, wst_ref, w2t_ref, o_ref):
    xb = x_ref[...].astype(jnp.bfloat16)
    y = jnp.dot(xb, wst_ref[...], preferred_element_type=jnp.float32) - b_ref[...]
    h = jnp.maximum(y, 0.0).astype(jnp.bfloat16)
    o_ref[...] = jnp.dot(h, w2t_ref[...], preferred_element_type=jnp.float32)


def kernel(x, csr_idx, W1, W2):
    N, D = x.shape
    G = csr_idx.shape[0] - 1
    GW = (((G + NW - 1) // NW) + 15) // 16 * 16
    GP = NW * GW

    csr32 = csr_idx.astype(jnp.int32)
    csr_pad = jnp.concatenate(
        [csr32, jnp.full((GP + 48 - (G + 1),), N, jnp.int32)]
    )
    x_flat = x.reshape(N * D)

    seg_minmax = pl.kernel(
        functools.partial(_seg_minmax_body, N, D, GW),
        out_type=(
            jax.ShapeDtypeStruct((GP * D,), jnp.float32),
            jax.ShapeDtypeStruct((GP * D,), jnp.float32),
        ),
        mesh=_mesh(),
        compiler_params=pltpu.CompilerParams(needs_layout_passes=False),
        scratch_types=(
            pltpu.VMEM((GW + 48,), jnp.int32),
            pltpu.VMEM((GW + 16,), jnp.int32),
            pltpu.VMEM((2 * CHUNK * D,), jnp.float32),
            pltpu.VMEM((GW * D,), jnp.float32),
            pltpu.VMEM((GW * D,), jnp.float32),
            pltpu.SemaphoreType.DMA((2,)),
        ),
    )
    mn, mx = seg_minmax(x_flat, csr_pad)

    pb, wst, w2t = pl.pallas_call(
        functools.partial(_mid_body, D),
        out_shape=(
            jax.ShapeDtypeStruct((GP, D), jnp.float32),
            jax.ShapeDtypeStruct((D, D), jnp.bfloat16),
            jax.ShapeDtypeStruct((D, D), jnp.bfloat16),
        ),
    )(mn.reshape(GP, D), mx.reshape(GP, D), W1, W2)

    expand = pl.kernel(
        functools.partial(_expand_body, N, D, GP),
        out_type=jax.ShapeDtypeStruct((N, D), jnp.float32),
        mesh=_mesh(),
        compiler_params=pltpu.CompilerParams(needs_layout_passes=False),
        scratch_types=(
            pltpu.VMEM((GP + 48,), jnp.int32),
            pltpu.VMEM((GP + 16,), jnp.int32),
            pltpu.VMEM((2 * CHUNK,), jnp.int32),
            pltpu.VMEM((2, CHUNK, D), jnp.float32),
            pltpu.SemaphoreType.DMA,
            pltpu.SemaphoreType.DMA((2,)),
        ),
    )
    bias = expand(pb, csr_pad)

    BR = 1280
    out = pl.pallas_call(
        _mlp_body,
        grid=(N // BR,),
        in_specs=[
            pl.BlockSpec((BR, D), lambda i: (i, 0)),
            pl.BlockSpec((BR, D), lambda i: (i, 0)),
            pl.BlockSpec((D, D), lambda i: (0, 0)),
            pl.BlockSpec((D, D), lambda i: (0, 0)),
        ],
        out_specs=pl.BlockSpec((BR, D), lambda i: (i, 0)),
        out_shape=jax.ShapeDtypeStruct((N, D), jnp.float32),
    )(x, bias, wst, w2t)
    return out
